# Initial kernel scaffold; baseline (speedup 1.0000x reference)
import jax
import jax.numpy as jnp
from jax.experimental import pallas as pl


def kernel(x, edge_attr, params, edge_index):
    raise NotImplementedError("write your pallas kernel here")


# speedup vs baseline: 3.1165x; 3.1165x over previous
import functools

import jax
import jax.numpy as jnp
import numpy as np
from jax import lax
from jax.experimental import pallas as pl
from jax.experimental.pallas import tpu as pltpu
from jax.experimental.pallas import tpu_sc as plsc

N_NODES = 10000
N_EDGES = 160000
D = 128
BN_EPS = 1e-5
LN_EPS = 1e-5
_BN_SCALE = np.float32(1.0 / np.sqrt(1.0 + BN_EPS))
_INV_SQRT2 = np.float32(1.0 / np.sqrt(2.0))

_NC = 2
_NS = 16
_NW = _NC * _NS
_EPW = N_EDGES // _NW
_CH = 128
_NFULL = _EPW // _CH
_TAIL = _EPW - _NFULL * _CH
_RPT = N_NODES // _NS


def _gelu(x):
    return 0.5 * x * (1.0 + lax.erf(x * _INV_SQRT2))


def _bn(x, g, b):
    return (x * _BN_SCALE) * g + b


def _ln(x, g, b):
    mu = jnp.mean(x, axis=-1, keepdims=True)
    d = x - mu
    var = jnp.mean(d * d, axis=-1, keepdims=True)
    return d * lax.rsqrt(var + LN_EPS) * g + b


def _mm(a, b):
    return jax.lax.dot_general(a, b, (((1,), (0,)), ((), ())),
                               preferred_element_type=jnp.float32)


def _node_mlp(x, p, g1, be1, g2, be2, wpa, wpb):
    def body(x_ref, w1, b1, g1r, be1r, w2, b2, g2r, be2r, wpa_r, wpb_r,
             o_ref, ta_ref, tb_ref):
        h = _gelu(_bn(_mm(x_ref[...], w1[...]) + b1[...], g1r[...], be1r[...]))
        h = _gelu(_bn(_mm(h, w2[...]) + b2[...], g2r[...], be2r[...]))
        o_ref[...] = h
        ta_ref[...] = _mm(h, wpa_r[...])
        tb_ref[...] = _mm(h, wpb_r[...])

    out = jax.ShapeDtypeStruct((N_NODES, D), jnp.float32)
    return pl.pallas_call(
        body,
        out_shape=(out, out, out),
    )(x, p["l1"]["W"], p["l1"]["b"].reshape(1, D), g1, be1,
      p["l2"]["W"], p["l2"]["b"].reshape(1, D), g2, be2, wpa, wpb)


def _edge_mlp(edge_attr, pn, pcat, emb):
    E_T = 2000
    grid = (N_EDGES // E_T,)

    def body(a_ref, w1, b1, g1, be1, w2, b2, g2, be2,
             embcat_ref, g0c, be0c, wc, bc, g1c, be1c, o_ref):
        a = a_ref[...]
        numer = a[:, : 32 - 4]
        h = _gelu(_bn(_mm(numer, w1[...]) + b1[...], g1[...], be1[...]))
        en = _gelu(_bn(_mm(h, w2[...]) + b2[...], g2[...], be2[...]))
        cat = a[:, 32 - 4:].astype(jnp.int32)
        ohs = []
        for i in range(4):
            ci = cat[:, i][:, None]
            oh = (ci == lax.broadcasted_iota(jnp.int32, (E_T, 8), 1))
            ohs.append(oh.astype(jnp.float32))
        oh_all = jnp.concatenate(ohs, axis=-1)
        ec = _mm(oh_all, embcat_ref[...])
        ec = _gelu(_bn(ec, g0c[...], be0c[...]))
        ec = _gelu(_bn(_mm(ec, wc[...]) + bc[...], g1c[...], be1c[...]))
        o_ref[...] = jnp.concatenate([en, ec], axis=-1)

    embcat = jnp.zeros((32, 32), jnp.float32)
    for i in range(4):
        embcat = lax.dynamic_update_slice(embcat, emb[i], (8 * i, 8 * i))

    espec = pl.BlockSpec((E_T, 32), lambda i: (i, 0))
    ospec = pl.BlockSpec((E_T, D), lambda i: (i, 0))
    full = lambda *s: pl.BlockSpec(s, lambda i: tuple(0 for _ in s))
    return pl.pallas_call(
        body,
        grid=grid,
        in_specs=[espec,
                  full(28, 96), full(1, 96), full(1, 96), full(1, 96),
                  full(96, 96), full(1, 96), full(1, 96), full(1, 96),
                  full(32, 32), full(1, 32), full(1, 32),
                  full(32, 32), full(1, 32), full(1, 32), full(1, 32)],
        out_specs=ospec,
        out_shape=jax.ShapeDtypeStruct((N_EDGES, D), jnp.float32),
    )(edge_attr,
      pn["l1"]["W"], pn["l1"]["b"].reshape(1, 96), pn["n1"]["g"].reshape(1, 96), pn["n1"]["be"].reshape(1, 96),
      pn["l2"]["W"], pn["l2"]["b"].reshape(1, 96), pn["n2"]["g"].reshape(1, 96), pn["n2"]["be"].reshape(1, 96),
      embcat, pcat["n0"]["g"].reshape(1, 32), pcat["n0"]["be"].reshape(1, 32),
      pcat["l"]["W"], pcat["l"]["b"].reshape(1, 32),
      pcat["n1"]["g"].reshape(1, 32), pcat["n1"]["be"].reshape(1, 32))


def _edge_stage(edge_emb, s, w, b, g, be, residual):
    E_T = 2000
    grid = (N_EDGES // E_T,)

    def body(e_ref, s_ref, w_ref, b_ref, g_ref, be_ref, o_ref):
        e = e_ref[...]
        y = _gelu(_ln(_mm(e, w_ref[...]) + s_ref[...] + b_ref[...],
                      g_ref[...], be_ref[...]))
        o_ref[...] = e + y if residual else y

    espec = pl.BlockSpec((E_T, D), lambda i: (i, 0))
    full = lambda *sh: pl.BlockSpec(sh, lambda i: tuple(0 for _ in sh))
    return pl.pallas_call(
        body,
        grid=grid,
        in_specs=[espec, espec, full(D, D), full(1, D), full(1, D), full(1, D)],
        out_specs=espec,
        out_shape=jax.ShapeDtypeStruct((N_EDGES, D), jnp.float32),
    )(edge_emb, s, w, b.reshape(1, D), g.reshape(1, D), be.reshape(1, D))


def _node_update(node_emb, aggp, wn1, wn2, b, g, be, wpa, wpb):
    def body(n_ref, a_ref, w1, w2, b_ref, g_ref, be_ref, wpa_r, wpb_r,
             o_ref, ta_ref, tb_ref):
        n = n_ref[...]
        agg = a_ref[0] + a_ref[1]
        u = _gelu(_ln(_mm(n, w1[...]) + _mm(agg, w2[...]) + b_ref[...],
                      g_ref[...], be_ref[...]))
        nn = n + u
        o_ref[...] = nn
        ta_ref[...] = _mm(nn, wpa_r[...])
        tb_ref[...] = _mm(nn, wpb_r[...])

    out = jax.ShapeDtypeStruct((N_NODES, D), jnp.float32)
    return pl.pallas_call(body, out_shape=(out, out, out))(
        node_emb, aggp, wn1, wn2, b.reshape(1, D), g.reshape(1, D),
        be.reshape(1, D), wpa, wpb)


def _final_stage(edge_emb, edge_pre, s, w_e, b1, g1, be1, w2, b2):
    E_T = 2000
    grid = (N_EDGES // E_T,)

    def body(e_ref, ep_ref, s_ref, w_ref, b1_ref, g_ref, be_ref,
             w2_ref, b2_ref, o_ref):
        e = e_ref[...] + ep_ref[...]
        h = _gelu(_bn(_mm(e, w_ref[...]) + s_ref[...] + b1_ref[...],
                      g_ref[...], be_ref[...]))
        o_ref[...] = _mm(h, w2_ref[...]) + b2_ref[...]

    espec = pl.BlockSpec((E_T, D), lambda i: (i, 0))
    full = lambda *sh: pl.BlockSpec(sh, lambda i: tuple(0 for _ in sh))
    return pl.pallas_call(
        body,
        grid=grid,
        in_specs=[espec, espec, espec, full(D, D), full(1, D), full(1, D),
                  full(1, D), full(D, 3), full(1, 3)],
        out_specs=pl.BlockSpec((E_T, 3), lambda i: (i, 0)),
        out_shape=jax.ShapeDtypeStruct((N_EDGES, 3), jnp.float32),
    )(edge_emb, edge_pre, s, w_e, b1.reshape(1, D), g1.reshape(1, D),
      be1.reshape(1, D), w2, b2.reshape(1, 3))


@functools.cache
def _sc_kernels():
    mesh = plsc.VectorSubcoreMesh(
        core_axis_name="c", subcore_axis_name="s",
        num_cores=_NC, num_subcores=_NS)

    @functools.partial(
        pl.kernel,
        out_type=jax.ShapeDtypeStruct((N_EDGES, D), jnp.float32),
        mesh=mesh,
        scratch_types=[
            pltpu.VMEM((_CH,), jnp.int32),
            pltpu.VMEM((_CH,), jnp.int32),
            pltpu.VMEM((_TAIL,), jnp.int32),
            pltpu.VMEM((_TAIL,), jnp.int32),
            pltpu.VMEM((_CH, D), jnp.float32),
            pltpu.VMEM((_CH, D), jnp.float32),
            pltpu.VMEM((_TAIL, D), jnp.float32),
            pltpu.VMEM((_TAIL, D), jnp.float32),
            pltpu.SemaphoreType.DMA,
            pltpu.SemaphoreType.DMA,
        ],
    )
    def gather2add(ta_hbm, tb_hbm, ia_hbm, ib_hbm, out_hbm,
                   ia_v, ib_v, ia8_v, ib8_v, ba_v, bb_v, ba8_v, bb8_v,
                   sem_a, sem_b):
        c = lax.axis_index("c")
        s = lax.axis_index("s")
        base = (s * _NC + c) * _EPW

        def do_chunk(off, n, iav, ibv, bav, bbv):
            pltpu.sync_copy(ia_hbm.at[pl.ds(off, n)], iav)
            pltpu.sync_copy(ib_hbm.at[pl.ds(off, n)], ibv)
            cpa = pltpu.async_copy(ta_hbm.at[iav], bav, sem_a)
            cpb = pltpu.async_copy(tb_hbm.at[ibv], bbv, sem_b)
            cpa.wait()
            cpb.wait()

            def addrow(r, carry):
                for cc in range(D // 16):
                    sl = pl.ds(cc * 16, 16)
                    bav[r, sl] = bav[r, sl] + bbv[r, sl]
                return carry

            lax.fori_loop(0, n, addrow, 0)
            pltpu.sync_copy(bav, out_hbm.at[pl.ds(off, n)])

        def body(k, carry):
            do_chunk(base + k * _CH, _CH, ia_v, ib_v, ba_v, bb_v)
            return carry

        lax.fori_loop(0, _NFULL, body, 0)
        do_chunk(base + _NFULL * _CH, _TAIL, ia8_v, ib8_v, ba8_v, bb8_v)

    @functools.partial(
        pl.kernel,
        out_type=jax.ShapeDtypeStruct((_NC, N_NODES, D), jnp.float32),
        mesh=mesh,
        scratch_types=[
            pltpu.VMEM((_CH,), jnp.int32),
            pltpu.VMEM((_TAIL,), jnp.int32),
            pltpu.VMEM((_CH, D), jnp.float32),
            pltpu.VMEM((_TAIL, D), jnp.float32),
            pltpu.VMEM_SHARED((N_NODES, D), jnp.float32),
        ],
    )
    def segsum(m_hbm, dst_hbm, out_hbm, idx_v, idx8_v, buf_v, buf8_v, acc_sh):
        c = lax.axis_index("c")
        s = lax.axis_index("s")

        def zrow(r, carry):
            for cc in range(D // 16):
                buf_v[r, pl.ds(cc * 16, 16)] = jnp.zeros((16,), jnp.float32)
            return carry

        lax.fori_loop(0, _CH, zrow, 0)
        row0 = s * 624
        for k in range(4):
            pltpu.sync_copy(buf_v, acc_sh.at[pl.ds(row0 + k * _CH, _CH)])
        pltpu.sync_copy(buf_v.at[pl.ds(0, 112)],
                        acc_sh.at[pl.ds(row0 + 512, 112)])

        @pl.when(s == _NS - 1)
        def _zero_tail():
            pltpu.sync_copy(buf_v.at[pl.ds(0, 16)],
                            acc_sh.at[pl.ds(N_NODES - 16, 16)])

        plsc.subcore_barrier()

        base = c * (N_EDGES // _NC) + s * _EPW

        def body(k, carry):
            off = base + k * _CH
            pltpu.sync_copy(dst_hbm.at[pl.ds(off, _CH)], idx_v)
            pltpu.sync_copy(m_hbm.at[pl.ds(off, _CH)], buf_v)
            pltpu.sync_copy(buf_v, acc_sh.at[idx_v], add=True)
            return carry

        lax.fori_loop(0, _NFULL, body, 0)
        off = base + _NFULL * _CH
        pltpu.sync_copy(dst_hbm.at[pl.ds(off, _TAIL)], idx8_v)
        pltpu.sync_copy(m_hbm.at[pl.ds(off, _TAIL)], buf8_v)
        pltpu.sync_copy(buf8_v, acc_sh.at[idx8_v], add=True)
        plsc.subcore_barrier()

        def out_rows(r, n):
            pltpu.sync_copy(acc_sh.at[pl.ds(r, n)], buf_v.at[pl.ds(0, n)])
            pltpu.sync_copy(buf_v.at[pl.ds(0, n)], out_hbm.at[c, pl.ds(r, n)])

        for k in range(4):
            out_rows(row0 + k * _CH, _CH)
        out_rows(row0 + 512, 112)

        @pl.when(s == _NS - 1)
        def _out_tail():
            out_rows(N_NODES - 16, 16)

    return gather2add, segsum


def _sc_gather2add(ta, tb, ia, ib):
    return _sc_kernels()[0](ta, tb, ia, ib)


def _sc_segsum(m, dst):
    return _sc_kernels()[1](m, dst)


def kernel(x, edge_attr, params, edge_index):
    src = edge_index[0]
    dst = edge_index[1]

    gnn = params["gnn"]
    msgW = [lp["msg"]["W"] for lp in gnn]
    edgW = [lp["edge"]["W"] for lp in gnn]
    nodW = [lp["node"]["W"] for lp in gnn]
    finW = params["final"]["l1"]["W"]

    pm = params["node_mlp"]
    node_emb, ta, tb = _node_mlp(
        x, pm, pm["n1"]["g"].reshape(1, D), pm["n1"]["be"].reshape(1, D),
        pm["n2"]["g"].reshape(1, D), pm["n2"]["be"].reshape(1, D),
        msgW[0][0:D], msgW[0][D:2 * D])
    node_pre = node_emb

    edge_emb = _edge_mlp(edge_attr, params["edge_numer"],
                         params["edge_cat_mlp"], params["cat_emb"])
    edge_pre = edge_emb

    for l in range(3):
        lp = gnn[l]
        s1 = _sc_gather2add(ta, tb, dst, src)
        m = _edge_stage(edge_emb, s1, msgW[l][2 * D:3 * D], lp["msg"]["b"],
                        lp["msg"]["g"], lp["msg"]["be"], residual=False)
        aggp = _sc_segsum(m, dst)
        if l < 2:
            wna, wnb = msgW[l + 1][0:D], msgW[l + 1][D:2 * D]
        else:
            wna, wnb = finW[0:D], finW[D:2 * D]
        node_emb, ua, ub = _node_update(
            node_emb, aggp, nodW[l][0:D], nodW[l][D:2 * D], lp["node"]["b"],
            lp["node"]["g"], lp["node"]["be"],
            edgW[l][D:2 * D], edgW[l][2 * D:3 * D])
        s2 = _sc_gather2add(ua, ub, dst, src)
        edge_emb = _edge_stage(edge_emb, s2, edgW[l][0:D], lp["edge"]["b"],
                               lp["edge"]["g"], lp["edge"]["be"], residual=True)
        if l < 2:
            ta, tb = _node_proj(node_emb, wna, wnb)
        else:
            ta, tb = _node_proj_res(node_emb, node_pre, wna, wnb)

    s3 = _sc_gather2add(ta, tb, src, dst)
    pf = params["final"]
    return _final_stage(edge_emb, edge_pre, s3, finW[2 * D:3 * D],
                        pf["l1"]["b"], pf["n1"]["g"], pf["n1"]["be"],
                        pf["l2"]["W"], pf["l2"]["b"])


def _node_proj(node_emb, wa, wb):
    def body(n_ref, wa_r, wb_r, ta_ref, tb_ref):
        n = n_ref[...]
        ta_ref[...] = _mm(n, wa_r[...])
        tb_ref[...] = _mm(n, wb_r[...])

    out = jax.ShapeDtypeStruct((N_NODES, D), jnp.float32)
    return pl.pallas_call(body, out_shape=(out, out))(node_emb, wa, wb)


def _node_proj_res(node_emb, node_pre, wa, wb):
    def body(n_ref, p_ref, wa_r, wb_r, ta_ref, tb_ref):
        n = n_ref[...] + p_ref[...]
        ta_ref[...] = _mm(n, wa_r[...])
        tb_ref[...] = _mm(n, wb_r[...])

    out = jax.ShapeDtypeStruct((N_NODES, D), jnp.float32)
    return pl.pallas_call(body, out_shape=(out, out))(node_emb, node_pre, wa, wb)


# speedup vs baseline: 3.5193x; 1.1292x over previous
import functools

import jax
import jax.numpy as jnp
import numpy as np
from jax import lax
from jax.experimental import pallas as pl
from jax.experimental.pallas import tpu as pltpu
from jax.experimental.pallas import tpu_sc as plsc

N_NODES = 10000
N_EDGES = 160000
D = 128
BN_EPS = 1e-5
LN_EPS = 1e-5
_BN_SCALE = np.float32(1.0 / np.sqrt(1.0 + BN_EPS))
_INV_SQRT2 = np.float32(1.0 / np.sqrt(2.0))

_NC = 2
_NS = 16
_NW = _NC * _NS
_CH = 128
_CPW = 40
_NCHUNK = _NW * _CPW
E_PAD = _NCHUNK * _CH
_ACC = N_NODES + 16


def _gelu(x):
    return 0.5 * x * (1.0 + lax.erf(x * _INV_SQRT2))


def _bn(x, g, b):
    return (x * _BN_SCALE) * g + b


def _ln(x, g, b):
    mu = jnp.mean(x, axis=-1, keepdims=True)
    d = x - mu
    var = jnp.mean(d * d, axis=-1, keepdims=True)
    return d * lax.rsqrt(var + LN_EPS) * g + b


def _mm(a, b):
    return jax.lax.dot_general(a, b, (((1,), (0,)), ((), ())),
                               preferred_element_type=jnp.float32)


def _node_mlp(x, p, g1, be1, g2, be2, wpa, wpb):
    def body(x_ref, w1, b1, g1r, be1r, w2, b2, g2r, be2r, wpa_r, wpb_r,
             o_ref, ta_ref, tb_ref):
        h = _gelu(_bn(_mm(x_ref[...], w1[...]) + b1[...], g1r[...], be1r[...]))
        h = _gelu(_bn(_mm(h, w2[...]) + b2[...], g2r[...], be2r[...]))
        o_ref[...] = h
        ta_ref[...] = _mm(h, wpa_r[...])
        tb_ref[...] = _mm(h, wpb_r[...])

    out = jax.ShapeDtypeStruct((N_NODES, D), jnp.float32)
    return pl.pallas_call(
        body,
        out_shape=(out, out, out),
    )(x, p["l1"]["W"], p["l1"]["b"].reshape(1, D), g1, be1,
      p["l2"]["W"], p["l2"]["b"].reshape(1, D), g2, be2, wpa, wpb)


def _edge_mlp(edge_attr, pn, pcat, emb):
    E_T = 2048
    grid = (E_PAD // E_T,)

    def body(a_ref, w1, b1, g1, be1, w2, b2, g2, be2,
             embcat_ref, g0c, be0c, wc, bc, g1c, be1c, o_ref):
        a = a_ref[...]
        numer = a[:, : 32 - 4]
        h = _gelu(_bn(_mm(numer, w1[...]) + b1[...], g1[...], be1[...]))
        en = _gelu(_bn(_mm(h, w2[...]) + b2[...], g2[...], be2[...]))
        cat = a[:, 32 - 4:].astype(jnp.int32)
        ohs = []
        for i in range(4):
            ci = cat[:, i][:, None]
            oh = (ci == lax.broadcasted_iota(jnp.int32, (E_T, 8), 1))
            ohs.append(oh.astype(jnp.float32))
        oh_all = jnp.concatenate(ohs, axis=-1)
        ec = _mm(oh_all, embcat_ref[...])
        ec = _gelu(_bn(ec, g0c[...], be0c[...]))
        ec = _gelu(_bn(_mm(ec, wc[...]) + bc[...], g1c[...], be1c[...]))
        o_ref[...] = jnp.concatenate([en, ec], axis=-1)

    embcat = jnp.zeros((32, 32), jnp.float32)
    for i in range(4):
        embcat = lax.dynamic_update_slice(embcat, emb[i], (8 * i, 8 * i))

    espec = pl.BlockSpec((E_T, 32), lambda i: (i, 0))
    ospec = pl.BlockSpec((E_T, D), lambda i: (i, 0))
    full = lambda *s: pl.BlockSpec(s, lambda i: tuple(0 for _ in s))
    return pl.pallas_call(
        body,
        grid=grid,
        in_specs=[espec,
                  full(28, 96), full(1, 96), full(1, 96), full(1, 96),
                  full(96, 96), full(1, 96), full(1, 96), full(1, 96),
                  full(32, 32), full(1, 32), full(1, 32),
                  full(32, 32), full(1, 32), full(1, 32), full(1, 32)],
        out_specs=ospec,
        out_shape=jax.ShapeDtypeStruct((E_PAD, D), jnp.float32),
    )(edge_attr,
      pn["l1"]["W"], pn["l1"]["b"].reshape(1, 96), pn["n1"]["g"].reshape(1, 96), pn["n1"]["be"].reshape(1, 96),
      pn["l2"]["W"], pn["l2"]["b"].reshape(1, 96), pn["n2"]["g"].reshape(1, 96), pn["n2"]["be"].reshape(1, 96),
      embcat, pcat["n0"]["g"].reshape(1, 32), pcat["n0"]["be"].reshape(1, 32),
      pcat["l"]["W"], pcat["l"]["b"].reshape(1, 32),
      pcat["n1"]["g"].reshape(1, 32), pcat["n1"]["be"].reshape(1, 32))


def _edge_stage(edge_emb, s, w, b, g, be, residual):
    E_T = 2048
    grid = (E_PAD // E_T,)

    def body(e_ref, s_ref, w_ref, b_ref, g_ref, be_ref, o_ref):
        e = e_ref[...]
        y = _gelu(_ln(_mm(e, w_ref[...]) + s_ref[...] + b_ref[...],
                      g_ref[...], be_ref[...]))
        o_ref[...] = e + y if residual else y

    espec = pl.BlockSpec((E_T, D), lambda i: (i, 0))
    full = lambda *sh: pl.BlockSpec(sh, lambda i: tuple(0 for _ in sh))
    return pl.pallas_call(
        body,
        grid=grid,
        in_specs=[espec, espec, full(D, D), full(1, D), full(1, D), full(1, D)],
        out_specs=espec,
        out_shape=jax.ShapeDtypeStruct((E_PAD, D), jnp.float32),
    )(edge_emb, s, w, b.reshape(1, D), g.reshape(1, D), be.reshape(1, D))


def _node_update(node_emb, aggp, wn1, wn2, b, g, be, wpa, wpb):
    def body(n_ref, a_ref, w1, w2, b_ref, g_ref, be_ref, wpa_r, wpb_r,
             o_ref, ta_ref, tb_ref):
        n = n_ref[...]
        agg = a_ref[0] + a_ref[1]
        u = _gelu(_ln(_mm(n, w1[...]) + _mm(agg, w2[...]) + b_ref[...],
                      g_ref[...], be_ref[...]))
        nn = n + u
        o_ref[...] = nn
        ta_ref[...] = _mm(nn, wpa_r[...])
        tb_ref[...] = _mm(nn, wpb_r[...])

    out = jax.ShapeDtypeStruct((N_NODES, D), jnp.float32)
    return pl.pallas_call(body, out_shape=(out, out, out))(
        node_emb, aggp, wn1, wn2, b.reshape(1, D), g.reshape(1, D),
        be.reshape(1, D), wpa, wpb)


def _final_stage(edge_emb, edge_pre, s, w_e, b1, g1, be1, w2, b2):
    E_T = 2048
    grid = (E_PAD // E_T,)

    def body(e_ref, ep_ref, s_ref, w_ref, b1_ref, g_ref, be_ref,
             w2_ref, b2_ref, o_ref):
        e = e_ref[...] + ep_ref[...]
        h = _gelu(_bn(_mm(e, w_ref[...]) + s_ref[...] + b1_ref[...],
                      g_ref[...], be_ref[...]))
        o_ref[...] = _mm(h, w2_ref[...]) + b2_ref[...]

    espec = pl.BlockSpec((E_T, D), lambda i: (i, 0))
    full = lambda *sh: pl.BlockSpec(sh, lambda i: tuple(0 for _ in sh))
    return pl.pallas_call(
        body,
        grid=grid,
        in_specs=[espec, espec, espec, full(D, D), full(1, D), full(1, D),
                  full(1, D), full(D, 3), full(1, 3)],
        out_specs=pl.BlockSpec((E_T, 3), lambda i: (i, 0)),
        out_shape=jax.ShapeDtypeStruct((E_PAD, 3), jnp.float32),
    )(edge_emb, edge_pre, s, w_e, b1.reshape(1, D), g1.reshape(1, D),
      be1.reshape(1, D), w2, b2.reshape(1, 3))


@functools.cache
def _sc_kernels():
    mesh = plsc.VectorSubcoreMesh(
        core_axis_name="c", subcore_axis_name="s",
        num_cores=_NC, num_subcores=_NS)


    @functools.partial(
        pl.kernel,
        out_type=jax.ShapeDtypeStruct((E_PAD, D), jnp.float32),
        mesh=mesh,
        scratch_types=[
            pltpu.VMEM((_CPW, _CH), jnp.int32),
            pltpu.VMEM((_CPW, _CH), jnp.int32),
            pltpu.VMEM((_CH, D), jnp.float32),
            pltpu.VMEM((_CH, D), jnp.float32),
            pltpu.VMEM((_CH, D), jnp.float32),
            pltpu.VMEM((_CH, D), jnp.float32),
            pltpu.SemaphoreType.DMA,
            pltpu.SemaphoreType.DMA,
            pltpu.SemaphoreType.DMA,
            pltpu.SemaphoreType.DMA,
            pltpu.SemaphoreType.DMA,
        ],
    )
    def gather2add(ta_hbm, tb_hbm, ia_hbm, ib_hbm, out_hbm,
                   ia_v, ib_v, ba0, bb0, ba1, bb1,
                   sa0, sb0, sa1, sb1, so):
        c = lax.axis_index("c")
        s = lax.axis_index("s")
        w = s * _NC + c
        first = w * _CPW
        pltpu.sync_copy(ia_hbm.at[pl.ds(first, _CPW)], ia_v)
        pltpu.sync_copy(ib_hbm.at[pl.ds(first, _CPW)], ib_v)

        def start(k, ba, bb, sa, sb):
            pltpu.async_copy(ta_hbm.at[ia_v.at[k]], ba, sa)
            pltpu.async_copy(tb_hbm.at[ib_v.at[k]], bb, sb)

        def finish(k, ba, bb, sa, sb):
            pltpu.make_async_copy(ta_hbm.at[ia_v.at[k]], ba, sa).wait()
            pltpu.make_async_copy(tb_hbm.at[ib_v.at[k]], bb, sb).wait()

            def addrow(r, carry):
                for cc in range(D // 16):
                    sl = pl.ds(cc * 16, 16)
                    ba[r, sl] = ba[r, sl] + bb[r, sl]
                return carry

            lax.fori_loop(0, _CH, addrow, 0)
            pltpu.async_copy(ba, out_hbm.at[pl.ds((first + k) * _CH, _CH)], so)

        def drain_out():
            pltpu.make_async_copy(ba0, out_hbm.at[pl.ds(0, _CH)], so).wait()

        def pair(j, carry):
            k0 = 2 * j
            k1 = 2 * j + 1
            start(k0, ba0, bb0, sa0, sb0)
            start(k1, ba1, bb1, sa1, sb1)
            finish(k0, ba0, bb0, sa0, sb0)
            finish(k1, ba1, bb1, sa1, sb1)
            drain_out()
            drain_out()
            return carry

        lax.fori_loop(0, _CPW // 2, pair, 0)

    @functools.partial(
        pl.kernel,
        out_type=jax.ShapeDtypeStruct((_NC, N_NODES, D), jnp.float32),
        mesh=mesh,
        scratch_types=[
            pltpu.VMEM((_CPW, _CH), jnp.int32),
            pltpu.VMEM((_CH, D), jnp.float32),
            pltpu.VMEM((_CH, D), jnp.float32),
            pltpu.VMEM_SHARED((_ACC, D), jnp.float32),
            pltpu.SemaphoreType.DMA,
            pltpu.SemaphoreType.DMA,
            pltpu.SemaphoreType.DMA,
        ],
    )
    def segsum(m_hbm, dst_hbm, out_hbm, idx_v, bm0, bm1, acc_sh,
               sm0, sm1, ss):
        c = lax.axis_index("c")
        s = lax.axis_index("s")
        w = s * _NC + c
        first = w * _CPW

        def zrow(r, carry):
            for cc in range(D // 16):
                bm0[r, pl.ds(cc * 16, 16)] = jnp.zeros((16,), jnp.float32)
            return carry

        lax.fori_loop(0, _CH, zrow, 0)
        row0 = s * 624
        for k in range(4):
            pltpu.sync_copy(bm0, acc_sh.at[pl.ds(row0 + k * _CH, _CH)])
        pltpu.sync_copy(bm0.at[pl.ds(0, 112)],
                        acc_sh.at[pl.ds(row0 + 512, 112)])

        @pl.when(s == _NS - 1)
        def _zero_tail():
            pltpu.sync_copy(bm0.at[pl.ds(0, 32)],
                            acc_sh.at[pl.ds(_ACC - 32, 32)])

        pltpu.sync_copy(dst_hbm.at[pl.ds(first, _CPW)], idx_v)
        plsc.subcore_barrier()

        def load(k, bm, sm):
            pltpu.async_copy(m_hbm.at[pl.ds((first + k) * _CH, _CH)], bm, sm)

        def scat(k, bm, sm):
            pltpu.make_async_copy(m_hbm.at[pl.ds(0, _CH)], bm, sm).wait()
            pltpu.async_copy(bm, acc_sh.at[idx_v.at[k]], ss, add=True)

        def drain_scat():
            pltpu.make_async_copy(bm0, acc_sh.at[pl.ds(0, _CH)], ss).wait()

        def pair(j, carry):
            k0 = 2 * j
            k1 = 2 * j + 1
            load(k0, bm0, sm0)
            load(k1, bm1, sm1)
            scat(k0, bm0, sm0)
            scat(k1, bm1, sm1)
            drain_scat()
            drain_scat()
            return carry

        lax.fori_loop(0, _CPW // 2, pair, 0)
        plsc.subcore_barrier()

        def out_rows(r, n):
            pltpu.sync_copy(acc_sh.at[pl.ds(r, n)], bm0.at[pl.ds(0, n)])
            pltpu.sync_copy(bm0.at[pl.ds(0, n)], out_hbm.at[c, pl.ds(r, n)])

        for k in range(4):
            out_rows(row0 + k * _CH, _CH)
        out_rows(row0 + 512, 112)

        @pl.when(s == _NS - 1)
        def _out_tail():
            out_rows(N_NODES - 16, 16)

    return gather2add, segsum


def _sc_gather2add(ta, tb, ia, ib):
    return _sc_kernels()[0](ta, tb, ia, ib)


def _sc_segsum(m, dst):
    return _sc_kernels()[1](m, dst)


def kernel(x, edge_attr, params, edge_index):
    pad_n = E_PAD - N_EDGES
    spread = (jnp.arange(pad_n, dtype=jnp.int32) * 37) % N_NODES
    src = jnp.concatenate([edge_index[0], spread]).reshape(_NCHUNK, _CH)
    dst = jnp.concatenate([edge_index[1], spread]).reshape(_NCHUNK, _CH)
    dst_s = jnp.concatenate(
        [edge_index[1],
         N_NODES + (jnp.arange(pad_n, dtype=jnp.int32) % 16)]
    ).reshape(_NCHUNK, _CH)
    edge_attr = jnp.pad(edge_attr, ((0, pad_n), (0, 0)))

    gnn = params["gnn"]
    msgW = [lp["msg"]["W"] for lp in gnn]
    edgW = [lp["edge"]["W"] for lp in gnn]
    nodW = [lp["node"]["W"] for lp in gnn]
    finW = params["final"]["l1"]["W"]

    pm = params["node_mlp"]
    node_emb, ta, tb = _node_mlp(
        x, pm, pm["n1"]["g"].reshape(1, D), pm["n1"]["be"].reshape(1, D),
        pm["n2"]["g"].reshape(1, D), pm["n2"]["be"].reshape(1, D),
        msgW[0][0:D], msgW[0][D:2 * D])
    node_pre = node_emb

    edge_emb = _edge_mlp(edge_attr, params["edge_numer"],
                         params["edge_cat_mlp"], params["cat_emb"])
    edge_pre = edge_emb

    for l in range(3):
        lp = gnn[l]
        s1 = _sc_gather2add(ta, tb, dst, src)
        m = _edge_stage(edge_emb, s1, msgW[l][2 * D:3 * D], lp["msg"]["b"],
                        lp["msg"]["g"], lp["msg"]["be"], residual=False)
        aggp = _sc_segsum(m, dst_s)
        if l < 2:
            wna, wnb = msgW[l + 1][0:D], msgW[l + 1][D:2 * D]
        else:
            wna, wnb = finW[0:D], finW[D:2 * D]
        node_emb, ua, ub = _node_update(
            node_emb, aggp, nodW[l][0:D], nodW[l][D:2 * D], lp["node"]["b"],
            lp["node"]["g"], lp["node"]["be"],
            edgW[l][D:2 * D], edgW[l][2 * D:3 * D])
        s2 = _sc_gather2add(ua, ub, dst, src)
        edge_emb = _edge_stage(edge_emb, s2, edgW[l][0:D], lp["edge"]["b"],
                               lp["edge"]["g"], lp["edge"]["be"], residual=True)
        if l < 2:
            ta, tb = _node_proj(node_emb, wna, wnb)
        else:
            ta, tb = _node_proj_res(node_emb, node_pre, wna, wnb)

    s3 = _sc_gather2add(ta, tb, src, dst)
    pf = params["final"]
    out = _final_stage(edge_emb, edge_pre, s3, finW[2 * D:3 * D],
                       pf["l1"]["b"], pf["n1"]["g"], pf["n1"]["be"],
                       pf["l2"]["W"], pf["l2"]["b"])
    return out[:N_EDGES]


def _node_proj(node_emb, wa, wb):
    def body(n_ref, wa_r, wb_r, ta_ref, tb_ref):
        n = n_ref[...]
        ta_ref[...] = _mm(n, wa_r[...])
        tb_ref[...] = _mm(n, wb_r[...])

    out = jax.ShapeDtypeStruct((N_NODES, D), jnp.float32)
    return pl.pallas_call(body, out_shape=(out, out))(node_emb, wa, wb)


def _node_proj_res(node_emb, node_pre, wa, wb):
    def body(n_ref, p_ref, wa_r, wb_r, ta_ref, tb_ref):
        n = n_ref[...] + p_ref[...]
        ta_ref[...] = _mm(n, wa_r[...])
        tb_ref[...] = _mm(n, wb_r[...])

    out = jax.ShapeDtypeStruct((N_NODES, D), jnp.float32)
    return pl.pallas_call(body, out_shape=(out, out))(node_emb, node_pre, wa, wb)


# speedup vs baseline: 3.8045x; 1.0811x over previous
import functools

import jax
import jax.numpy as jnp
import numpy as np
from jax import lax
from jax.experimental import pallas as pl
from jax.experimental.pallas import tpu as pltpu
from jax.experimental.pallas import tpu_sc as plsc

N_NODES = 10000
N_EDGES = 160000
D = 128
BN_EPS = 1e-5
LN_EPS = 1e-5
_BN_SCALE = np.float32(1.0 / np.sqrt(1.0 + BN_EPS))
_INV_SQRT2 = np.float32(1.0 / np.sqrt(2.0))

_NC = 2
_NS = 16
_NW = _NC * _NS
_CH = 128
_CPW = 40
_NCHUNK = _NW * _CPW
E_PAD = _NCHUNK * _CH
_ACC = N_NODES + 16


def _gelu(x):
    return 0.5 * x * (1.0 + lax.erf(x * _INV_SQRT2))


def _bn(x, g, b):
    return (x * _BN_SCALE) * g + b


def _ln(x, g, b):
    mu = jnp.mean(x, axis=-1, keepdims=True)
    d = x - mu
    var = jnp.mean(d * d, axis=-1, keepdims=True)
    return d * lax.rsqrt(var + LN_EPS) * g + b


def _mm(a, b):
    return jax.lax.dot_general(a, b, (((1,), (0,)), ((), ())),
                               preferred_element_type=jnp.float32)


def _node_mlp(x, p, g1, be1, g2, be2, wpa, wpb):
    def body(x_ref, w1, b1, g1r, be1r, w2, b2, g2r, be2r, wpa_r, wpb_r,
             o_ref, ta_ref, tb_ref):
        h = _gelu(_bn(_mm(x_ref[...], w1[...]) + b1[...], g1r[...], be1r[...]))
        h = _gelu(_bn(_mm(h, w2[...]) + b2[...], g2r[...], be2r[...]))
        o_ref[...] = h
        ta_ref[...] = _mm(h, wpa_r[...])
        tb_ref[...] = _mm(h, wpb_r[...])

    out = jax.ShapeDtypeStruct((N_NODES, D), jnp.float32)
    return pl.pallas_call(
        body,
        out_shape=(out, out, out),
    )(x, p["l1"]["W"], p["l1"]["b"].reshape(1, D), g1, be1,
      p["l2"]["W"], p["l2"]["b"].reshape(1, D), g2, be2, wpa, wpb)


def _edge_mlp(edge_attr, pn, pcat, emb):
    E_T = 2048
    grid = (E_PAD // E_T,)

    def body(a_ref, w1, b1, g1, be1, w2, b2, g2, be2,
             embcat_ref, g0c, be0c, wc, bc, g1c, be1c, o_ref):
        a = a_ref[...]
        numer = a[:, : 32 - 4]
        h = _gelu(_bn(_mm(numer, w1[...]) + b1[...], g1[...], be1[...]))
        en = _gelu(_bn(_mm(h, w2[...]) + b2[...], g2[...], be2[...]))
        cat = a[:, 32 - 4:].astype(jnp.int32)
        ohs = []
        for i in range(4):
            ci = cat[:, i][:, None]
            oh = (ci == lax.broadcasted_iota(jnp.int32, (E_T, 8), 1))
            ohs.append(oh.astype(jnp.float32))
        oh_all = jnp.concatenate(ohs, axis=-1)
        ec = _mm(oh_all, embcat_ref[...])
        ec = _gelu(_bn(ec, g0c[...], be0c[...]))
        ec = _gelu(_bn(_mm(ec, wc[...]) + bc[...], g1c[...], be1c[...]))
        o_ref[...] = jnp.concatenate([en, ec], axis=-1)

    embcat = jnp.zeros((32, 32), jnp.float32)
    for i in range(4):
        embcat = lax.dynamic_update_slice(embcat, emb[i], (8 * i, 8 * i))

    espec = pl.BlockSpec((E_T, 32), lambda i: (i, 0))
    ospec = pl.BlockSpec((E_T, D), lambda i: (i, 0))
    full = lambda *s: pl.BlockSpec(s, lambda i: tuple(0 for _ in s))
    return pl.pallas_call(
        body,
        grid=grid,
        in_specs=[espec,
                  full(28, 96), full(1, 96), full(1, 96), full(1, 96),
                  full(96, 96), full(1, 96), full(1, 96), full(1, 96),
                  full(32, 32), full(1, 32), full(1, 32),
                  full(32, 32), full(1, 32), full(1, 32), full(1, 32)],
        out_specs=ospec,
        out_shape=jax.ShapeDtypeStruct((E_PAD, D), jnp.float32),
    )(edge_attr,
      pn["l1"]["W"], pn["l1"]["b"].reshape(1, 96), pn["n1"]["g"].reshape(1, 96), pn["n1"]["be"].reshape(1, 96),
      pn["l2"]["W"], pn["l2"]["b"].reshape(1, 96), pn["n2"]["g"].reshape(1, 96), pn["n2"]["be"].reshape(1, 96),
      embcat, pcat["n0"]["g"].reshape(1, 32), pcat["n0"]["be"].reshape(1, 32),
      pcat["l"]["W"], pcat["l"]["b"].reshape(1, 32),
      pcat["n1"]["g"].reshape(1, 32), pcat["n1"]["be"].reshape(1, 32))


def _edge_stage(edge_emb, s, w, b, g, be, residual):
    E_T = 2048
    grid = (E_PAD // E_T,)

    def body(e_ref, s_ref, w_ref, b_ref, g_ref, be_ref, o_ref):
        e = e_ref[...]
        y = _gelu(_ln(_mm(e, w_ref[...]) + s_ref[...] + b_ref[...],
                      g_ref[...], be_ref[...]))
        o_ref[...] = e + y if residual else y

    espec = pl.BlockSpec((E_T, D), lambda i: (i, 0))
    full = lambda *sh: pl.BlockSpec(sh, lambda i: tuple(0 for _ in sh))
    return pl.pallas_call(
        body,
        grid=grid,
        in_specs=[espec, espec, full(D, D), full(1, D), full(1, D), full(1, D)],
        out_specs=espec,
        out_shape=jax.ShapeDtypeStruct((E_PAD, D), jnp.float32),
    )(edge_emb, s, w, b.reshape(1, D), g.reshape(1, D), be.reshape(1, D))


def _node_update(node_emb, aggp, wn1, wn2, b, g, be, wpa, wpb):
    def body(n_ref, a_ref, w1, w2, b_ref, g_ref, be_ref, wpa_r, wpb_r,
             o_ref, ta_ref, tb_ref):
        n = n_ref[...]
        agg = a_ref[0] + a_ref[1]
        u = _gelu(_ln(_mm(n, w1[...]) + _mm(agg, w2[...]) + b_ref[...],
                      g_ref[...], be_ref[...]))
        nn = n + u
        o_ref[...] = nn
        ta_ref[...] = _mm(nn, wpa_r[...])
        tb_ref[...] = _mm(nn, wpb_r[...])

    out = jax.ShapeDtypeStruct((N_NODES, D), jnp.float32)
    return pl.pallas_call(body, out_shape=(out, out, out))(
        node_emb, aggp, wn1, wn2, b.reshape(1, D), g.reshape(1, D),
        be.reshape(1, D), wpa, wpb)


def _final_stage(edge_emb, edge_pre, s, w_e, b1, g1, be1, w2, b2):
    E_T = 2048
    grid = (E_PAD // E_T,)

    def body(e_ref, ep_ref, s_ref, w_ref, b1_ref, g_ref, be_ref,
             w2_ref, b2_ref, o_ref):
        e = e_ref[...] + ep_ref[...]
        h = _gelu(_bn(_mm(e, w_ref[...]) + s_ref[...] + b1_ref[...],
                      g_ref[...], be_ref[...]))
        o_ref[...] = _mm(h, w2_ref[...]) + b2_ref[...]

    espec = pl.BlockSpec((E_T, D), lambda i: (i, 0))
    full = lambda *sh: pl.BlockSpec(sh, lambda i: tuple(0 for _ in sh))
    return pl.pallas_call(
        body,
        grid=grid,
        in_specs=[espec, espec, espec, full(D, D), full(1, D), full(1, D),
                  full(1, D), full(D, 3), full(1, 3)],
        out_specs=pl.BlockSpec((E_T, 3), lambda i: (i, 0)),
        out_shape=jax.ShapeDtypeStruct((E_PAD, 3), jnp.float32),
    )(edge_emb, edge_pre, s, w_e, b1.reshape(1, D), g1.reshape(1, D),
      be1.reshape(1, D), w2, b2.reshape(1, 3))


@functools.cache
def _sc_kernels():
    mesh = plsc.VectorSubcoreMesh(
        core_axis_name="c", subcore_axis_name="s",
        num_cores=_NC, num_subcores=_NS)


    @functools.partial(
        pl.kernel,
        out_type=jax.ShapeDtypeStruct((E_PAD, D), jnp.float32),
        mesh=mesh,
        scratch_types=[
            pltpu.VMEM((_CPW, _CH), jnp.int32),
            pltpu.VMEM((_CPW, _CH), jnp.int32),
            pltpu.VMEM((_CH, D), jnp.float32),
            pltpu.VMEM((_CH, D), jnp.float32),
            pltpu.VMEM((_CH, D), jnp.float32),
            pltpu.VMEM((_CH, D), jnp.float32),
            pltpu.VMEM((_CH, D), jnp.float32),
            pltpu.VMEM((_CH, D), jnp.float32),
        ] + [pltpu.SemaphoreType.DMA] * 9,
    )
    def gather2add(ta_hbm, tb_hbm, ia_hbm, ib_hbm, out_hbm,
                   ia_v, ib_v, ba0, bb0, ba1, bb1, ba2, bb2,
                   sa0, sb0, sa1, sb1, sa2, sb2, so0, so1, so2):
        c = lax.axis_index("c")
        s = lax.axis_index("s")
        w = s * _NC + c
        first = w * _CPW
        pltpu.sync_copy(ia_hbm.at[pl.ds(first, _CPW)], ia_v)
        pltpu.sync_copy(ib_hbm.at[pl.ds(first, _CPW)], ib_v)

        ba = [ba0, ba1, ba2]
        bb = [bb0, bb1, bb2]
        sa = [sa0, sa1, sa2]
        sb = [sb0, sb1, sb2]
        so = [so0, so1, so2]

        def start(k):
            t = k % 3
            pltpu.async_copy(ta_hbm.at[ia_v.at[k]], ba[t], sa[t])
            pltpu.async_copy(tb_hbm.at[ib_v.at[k]], bb[t], sb[t])

        def finish(k):
            t = k % 3
            pltpu.make_async_copy(ta_hbm.at[ia_v.at[k]], ba[t], sa[t]).wait()
            pltpu.make_async_copy(tb_hbm.at[ib_v.at[k]], bb[t], sb[t]).wait()
            bav, bbv = ba[t], bb[t]

            def addrow(r, carry):
                for cc in range(D // 16):
                    sl = pl.ds(cc * 16, 16)
                    plsc.addupdate(bav.at[r, sl], bbv[r, sl])
                return carry

            lax.fori_loop(0, _CH, addrow, 0)
            pltpu.async_copy(bav, out_hbm.at[pl.ds((first + k) * _CH, _CH)],
                             so[t])

        def drain_out(k):
            t = k % 3
            pltpu.make_async_copy(ba[t], out_hbm.at[pl.ds(0, _CH)],
                                  so[t]).wait()

        for k in range(_CPW + 2):
            if k < _CPW:
                if k >= 3:
                    drain_out(k - 3)
                start(k)
            if k >= 2:
                finish(k - 2)
        for k in range(_CPW - 3, _CPW):
            drain_out(k)

    @functools.partial(
        pl.kernel,
        out_type=jax.ShapeDtypeStruct((_NC, N_NODES, D), jnp.float32),
        mesh=mesh,
        scratch_types=[
            pltpu.VMEM((_CPW, _CH), jnp.int32),
            pltpu.VMEM((_CH, D), jnp.float32),
            pltpu.VMEM((_CH, D), jnp.float32),
            pltpu.VMEM_SHARED((_ACC, D), jnp.float32),
        ] + [pltpu.SemaphoreType.DMA] * 4,
    )
    def segsum(m_hbm, dst_hbm, out_hbm, idx_v, bm0, bm1, acc_sh,
               sm0, sm1, ss0, ss1):
        c = lax.axis_index("c")
        s = lax.axis_index("s")
        w = s * _NC + c
        first = w * _CPW

        def zrow(r, carry):
            for cc in range(D // 16):
                bm0[r, pl.ds(cc * 16, 16)] = jnp.zeros((16,), jnp.float32)
            return carry

        lax.fori_loop(0, _CH, zrow, 0)
        row0 = s * 624
        for k in range(4):
            pltpu.sync_copy(bm0, acc_sh.at[pl.ds(row0 + k * _CH, _CH)])
        pltpu.sync_copy(bm0.at[pl.ds(0, 112)],
                        acc_sh.at[pl.ds(row0 + 512, 112)])

        @pl.when(s == _NS - 1)
        def _zero_tail():
            pltpu.sync_copy(bm0.at[pl.ds(0, 32)],
                            acc_sh.at[pl.ds(_ACC - 32, 32)])

        pltpu.sync_copy(dst_hbm.at[pl.ds(first, _CPW)], idx_v)
        plsc.subcore_barrier()

        bm = [bm0, bm1]
        sm = [sm0, sm1]
        ss = [ss0, ss1]

        def load(k):
            t = k % 2
            pltpu.async_copy(m_hbm.at[pl.ds((first + k) * _CH, _CH)],
                             bm[t], sm[t])

        def scat(k):
            t = k % 2
            pltpu.make_async_copy(m_hbm.at[pl.ds(0, _CH)], bm[t], sm[t]).wait()
            pltpu.async_copy(bm[t], acc_sh.at[idx_v.at[k]], ss[t], add=True)

        def drain_scat(k):
            t = k % 2
            pltpu.make_async_copy(bm[t], acc_sh.at[pl.ds(0, _CH)],
                                  ss[t]).wait()

        for k in range(_CPW + 1):
            if k < _CPW:
                if k >= 2:
                    drain_scat(k - 2)
                load(k)
            if k >= 1:
                scat(k - 1)
        for k in range(_CPW - 2, _CPW):
            drain_scat(k)
        plsc.subcore_barrier()

        def out_rows(r, n):
            pltpu.sync_copy(acc_sh.at[pl.ds(r, n)], bm0.at[pl.ds(0, n)])
            pltpu.sync_copy(bm0.at[pl.ds(0, n)], out_hbm.at[c, pl.ds(r, n)])

        for k in range(4):
            out_rows(row0 + k * _CH, _CH)
        out_rows(row0 + 512, 112)

        @pl.when(s == _NS - 1)
        def _out_tail():
            out_rows(N_NODES - 16, 16)

    return gather2add, segsum


def _sc_gather2add(ta, tb, ia, ib):
    return _sc_kernels()[0](ta, tb, ia, ib)


def _sc_segsum(m, dst):
    return _sc_kernels()[1](m, dst)


def kernel(x, edge_attr, params, edge_index):
    pad_n = E_PAD - N_EDGES
    spread = (jnp.arange(pad_n, dtype=jnp.int32) * 37) % N_NODES
    src = jnp.concatenate([edge_index[0], spread]).reshape(_NCHUNK, _CH)
    dst = jnp.concatenate([edge_index[1], spread]).reshape(_NCHUNK, _CH)
    dst_s = jnp.concatenate(
        [edge_index[1],
         N_NODES + (jnp.arange(pad_n, dtype=jnp.int32) % 16)]
    ).reshape(_NCHUNK, _CH)
    edge_attr = jnp.pad(edge_attr, ((0, pad_n), (0, 0)))

    gnn = params["gnn"]
    msgW = [lp["msg"]["W"] for lp in gnn]
    edgW = [lp["edge"]["W"] for lp in gnn]
    nodW = [lp["node"]["W"] for lp in gnn]
    finW = params["final"]["l1"]["W"]

    pm = params["node_mlp"]
    node_emb, ta, tb = _node_mlp(
        x, pm, pm["n1"]["g"].reshape(1, D), pm["n1"]["be"].reshape(1, D),
        pm["n2"]["g"].reshape(1, D), pm["n2"]["be"].reshape(1, D),
        msgW[0][0:D], msgW[0][D:2 * D])
    node_pre = node_emb

    edge_emb = _edge_mlp(edge_attr, params["edge_numer"],
                         params["edge_cat_mlp"], params["cat_emb"])
    edge_pre = edge_emb

    for l in range(3):
        lp = gnn[l]
        s1 = _sc_gather2add(ta, tb, dst, src)
        m = _edge_stage(edge_emb, s1, msgW[l][2 * D:3 * D], lp["msg"]["b"],
                        lp["msg"]["g"], lp["msg"]["be"], residual=False)
        aggp = _sc_segsum(m, dst_s)
        if l < 2:
            wna, wnb = msgW[l + 1][0:D], msgW[l + 1][D:2 * D]
        else:
            wna, wnb = finW[0:D], finW[D:2 * D]
        node_emb, ua, ub = _node_update(
            node_emb, aggp, nodW[l][0:D], nodW[l][D:2 * D], lp["node"]["b"],
            lp["node"]["g"], lp["node"]["be"],
            edgW[l][D:2 * D], edgW[l][2 * D:3 * D])
        s2 = _sc_gather2add(ua, ub, dst, src)
        edge_emb = _edge_stage(edge_emb, s2, edgW[l][0:D], lp["edge"]["b"],
                               lp["edge"]["g"], lp["edge"]["be"], residual=True)
        if l < 2:
            ta, tb = _node_proj(node_emb, wna, wnb)
        else:
            ta, tb = _node_proj_res(node_emb, node_pre, wna, wnb)

    s3 = _sc_gather2add(ta, tb, src, dst)
    pf = params["final"]
    out = _final_stage(edge_emb, edge_pre, s3, finW[2 * D:3 * D],
                       pf["l1"]["b"], pf["n1"]["g"], pf["n1"]["be"],
                       pf["l2"]["W"], pf["l2"]["b"])
    return out[:N_EDGES]


def _node_proj(node_emb, wa, wb):
    def body(n_ref, wa_r, wb_r, ta_ref, tb_ref):
        n = n_ref[...]
        ta_ref[...] = _mm(n, wa_r[...])
        tb_ref[...] = _mm(n, wb_r[...])

    out = jax.ShapeDtypeStruct((N_NODES, D), jnp.float32)
    return pl.pallas_call(body, out_shape=(out, out))(node_emb, wa, wb)


def _node_proj_res(node_emb, node_pre, wa, wb):
    def body(n_ref, p_ref, wa_r, wb_r, ta_ref, tb_ref):
        n = n_ref[...] + p_ref[...]
        ta_ref[...] = _mm(n, wa_r[...])
        tb_ref[...] = _mm(n, wb_r[...])

    out = jax.ShapeDtypeStruct((N_NODES, D), jnp.float32)
    return pl.pallas_call(body, out_shape=(out, out))(node_emb, node_pre, wa, wb)
